# baseline (speedup 1.0000x reference)
import functools

import jax
import jax.numpy as jnp
from jax import lax
from jax.experimental import pallas as pl
from jax.experimental.pallas import tpu as pltpu
from jax.experimental.pallas import tpu_sc as plsc

BATCH = 16384
HIDDEN = 128
EMBED = 64


def _sc_gather(table, idx, batch=BATCH):
    info = plsc.get_sparse_core_info()
    nw = info.num_cores * info.num_subcores
    bpw = batch // nw
    mesh = plsc.VectorSubcoreMesh(core_axis_name="c", subcore_axis_name="s")

    @functools.partial(
        pl.kernel,
        mesh=mesh,
        out_type=jax.ShapeDtypeStruct((batch, HIDDEN), jnp.float32),
        scratch_types=[
            pltpu.VMEM((bpw,), jnp.int32),
            pltpu.VMEM((bpw, HIDDEN), jnp.float32),
            pltpu.SemaphoreType.DMA,
        ],
    )
    def k(table_hbm, idx_hbm, out_hbm, idx_v, rows_v, sem):
        wid = lax.axis_index("s") * info.num_cores + lax.axis_index("c")
        base = wid * bpw
        pltpu.sync_copy(idx_hbm.at[pl.ds(base, bpw)], idx_v)
        pltpu.async_copy(table_hbm.at[idx_v], rows_v, sem).wait()
        pltpu.sync_copy(rows_v, out_hbm.at[pl.ds(base, bpw)])

    return k(table, idx)


_BLK = 8192


def _mlp_body(g_ref, w1_ref, b1_ref, w2t_ref, b2_ref, out_ref):
    g = g_ref[...]
    h = jnp.dot(g, w1_ref[...], preferred_element_type=jnp.float32)
    h = jnp.maximum(h + b1_ref[...], 0.0)
    ot = lax.dot_general(w2t_ref[...], h, (((1,), (1,)), ((), ())),
                         preferred_element_type=jnp.float32)
    ot = ot + b2_ref[...].reshape(EMBED, 1)
    n2 = jnp.sum(ot * ot, axis=0, keepdims=True)
    out_ref[...] = ot * jnp.minimum(lax.rsqrt(n2), 1e12)


def _tc_mlp(g, W1, b1, W2t, b2, batch=BATCH, blk=_BLK):
    return pl.pallas_call(
        _mlp_body,
        grid=(batch // blk,),
        in_specs=[
            pl.BlockSpec((blk, HIDDEN), lambda i: (i, 0)),
            pl.BlockSpec((HIDDEN, HIDDEN), lambda i: (0, 0)),
            pl.BlockSpec((1, HIDDEN), lambda i: (0, 0)),
            pl.BlockSpec((EMBED, HIDDEN), lambda i: (0, 0)),
            pl.BlockSpec((1, EMBED), lambda i: (0, 0)),
        ],
        out_specs=pl.BlockSpec((EMBED, blk), lambda i: (0, i)),
        out_shape=jax.ShapeDtypeStruct((EMBED, batch), jnp.float32),
    )(g, W1, b1.reshape(1, HIDDEN), W2t, b2.reshape(1, EMBED))


def kernel(x, table, W1, b1, W2, b2):
    h = BATCH // 2
    W2t = W2.T
    g0 = _sc_gather(table, x[:h], h)
    g1 = _sc_gather(table, x[h:], h)
    o0 = _tc_mlp(g0, W1, b1, W2t, b2, h, 4096)
    o1 = _tc_mlp(g1, W1, b1, W2t, b2, h, 4096)
    return jnp.concatenate([o0, o1], axis=1).T


# speedup vs baseline: 1.2392x; 1.2392x over previous
import functools

import jax
import jax.numpy as jnp
from jax import lax
from jax.experimental import pallas as pl
from jax.experimental.pallas import tpu as pltpu
from jax.experimental.pallas import tpu_sc as plsc

BATCH = 16384
HIDDEN = 128
EMBED = 64


def _sc_gather(table, idx, batch=BATCH):
    info = plsc.get_sparse_core_info()
    nw = info.num_cores * info.num_subcores
    bpw = batch // nw
    mesh = plsc.VectorSubcoreMesh(core_axis_name="c", subcore_axis_name="s")

    @functools.partial(
        pl.kernel,
        mesh=mesh,
        out_type=jax.ShapeDtypeStruct((batch, HIDDEN), jnp.float32),
        scratch_types=[
            pltpu.VMEM((bpw,), jnp.int32),
            pltpu.VMEM((bpw, HIDDEN), jnp.float32),
            pltpu.SemaphoreType.DMA,
        ],
    )
    def k(table_hbm, idx_hbm, out_hbm, idx_v, rows_v, sem):
        wid = lax.axis_index("s") * info.num_cores + lax.axis_index("c")
        base = wid * bpw
        pltpu.sync_copy(idx_hbm.at[pl.ds(base, bpw)], idx_v)
        pltpu.async_copy(table_hbm.at[idx_v], rows_v, sem).wait()
        pltpu.sync_copy(rows_v, out_hbm.at[pl.ds(base, bpw)])

    return k(table, idx)


_BLK = 8192


def _mlp_body(g_ref, w1_ref, b1_ref, w2t_ref, b2_ref, out_ref):
    g = g_ref[...]
    h = jnp.dot(g, w1_ref[...], preferred_element_type=jnp.float32)
    h = jnp.maximum(h + b1_ref[...], 0.0)
    ot = lax.dot_general(w2t_ref[...], h, (((1,), (1,)), ((), ())),
                         preferred_element_type=jnp.float32)
    ot = ot + b2_ref[...].reshape(EMBED, 1)
    n2 = jnp.sum(ot * ot, axis=0, keepdims=True)
    out_ref[...] = ot * jnp.minimum(lax.rsqrt(n2), 1e12)


def _tc_mlp(g, W1, b1, W2t, b2, batch=BATCH, blk=_BLK):
    return pl.pallas_call(
        _mlp_body,
        grid=(batch // blk,),
        in_specs=[
            pl.BlockSpec((blk, HIDDEN), lambda i: (i, 0)),
            pl.BlockSpec((HIDDEN, HIDDEN), lambda i: (0, 0)),
            pl.BlockSpec((1, HIDDEN), lambda i: (0, 0)),
            pl.BlockSpec((EMBED, HIDDEN), lambda i: (0, 0)),
            pl.BlockSpec((1, EMBED), lambda i: (0, 0)),
        ],
        out_specs=pl.BlockSpec((EMBED, blk), lambda i: (0, i)),
        out_shape=jax.ShapeDtypeStruct((EMBED, batch), jnp.float32),
    )(g, W1, b1.reshape(1, HIDDEN), W2t, b2.reshape(1, EMBED))


def kernel(x, table, W1, b1, W2, b2):
    g = _sc_gather(table, x)
    out_t = _tc_mlp(g, W1, b1, W2.T, b2)
    return out_t.T
